# baseline (speedup 1.0000x reference)
import math

import jax
import jax.numpy as jnp
from jax import lax
from jax.experimental import pallas as pl

_T = 8192
_D = 2048
_E = 64
_K = 8
_BLOCK_T = 512
_EPS = 1e-5
_INV_SQRT2 = 1.0 / math.sqrt(2.0)


def _layernorm(h, gamma, beta):
    n = h.shape[-1]
    ones = jnp.full((n, 1), 1.0 / n, dtype=jnp.float32)
    mu = jnp.dot(h, ones, preferred_element_type=jnp.float32,
                 precision=lax.Precision.HIGHEST)
    d = h - mu
    var = jnp.dot(d * d, ones, preferred_element_type=jnp.float32,
                  precision=lax.Precision.HIGHEST)
    return d * jax.lax.rsqrt(var + _EPS) * gamma + beta


def _gelu_exact(h):
    return h * 0.5 * (1.0 + jax.lax.erf(h * _INV_SQRT2))


def _gate_kernel(x_ref, w1t_ref, b1_ref, g1_ref, be1_ref,
                 w2t_ref, b2_ref, g2_ref, be2_ref, w3t_ref,
                 w_ref, idx_ref, logits_ref):
    h = jnp.dot(x_ref[...], w1t_ref[...], preferred_element_type=jnp.float32)
    h = _gelu_exact(_layernorm(h + b1_ref[...], g1_ref[...], be1_ref[...]))
    h = jnp.dot(h, w2t_ref[...], preferred_element_type=jnp.float32)
    h = _gelu_exact(_layernorm(h + b2_ref[...], g2_ref[...], be2_ref[...]))
    logits = jnp.dot(h, w3t_ref[...], preferred_element_type=jnp.float32)
    logits_ref[...] = logits

    col = jax.lax.broadcasted_iota(jnp.int32, logits.shape, 1)
    revcol = (_E - 1 - col).astype(jnp.float32)
    work = logits
    vals = []
    idxs = []
    for _ in range(_K):
        m = jnp.max(work, axis=1, keepdims=True)
        hit = work == m
        rid = jnp.max(jnp.where(hit, revcol, -1.0), axis=1, keepdims=True)
        idxs.append(float(_E - 1) - rid)
        vals.append(m)
        work = jnp.where(hit, -jnp.inf, work)
    v = jnp.concatenate(vals, axis=1)
    iF = jnp.concatenate(idxs, axis=1)
    e = jnp.exp(v - v[:, 0:1])
    w_ref[...] = e / jnp.sum(e, axis=1, keepdims=True)
    idx_ref[...] = iF.astype(jnp.int32)


def kernel(x, w1, b1, g1, be1, w2, b2, g2, be2, w3):
    grid = (_T // _BLOCK_T,)
    tok = lambda i: (i, 0)
    rep = lambda i: (0, 0)
    out_shapes = (
        jax.ShapeDtypeStruct((_T, _K), jnp.float32),
        jax.ShapeDtypeStruct((_T, _K), jnp.int32),
        jax.ShapeDtypeStruct((_T, _E), jnp.float32),
    )
    f = pl.pallas_call(
        _gate_kernel,
        grid=grid,
        in_specs=[
            pl.BlockSpec((_BLOCK_T, _D), tok),
            pl.BlockSpec((_D, 128), rep),
            pl.BlockSpec((1, 128), rep),
            pl.BlockSpec((1, 128), rep),
            pl.BlockSpec((1, 128), rep),
            pl.BlockSpec((128, 32), rep),
            pl.BlockSpec((1, 32), rep),
            pl.BlockSpec((1, 32), rep),
            pl.BlockSpec((1, 32), rep),
            pl.BlockSpec((32, _E), rep),
        ],
        out_specs=(
            pl.BlockSpec((_BLOCK_T, _K), tok),
            pl.BlockSpec((_BLOCK_T, _K), tok),
            pl.BlockSpec((_BLOCK_T, _E), tok),
        ),
        out_shape=out_shapes,
    )
    return f(x, w1.T, b1[None, :], g1[None, :], be1[None, :],
             w2.T, b2[None, :], g2[None, :], be2[None, :], w3.T)


# speedup vs baseline: 1.9480x; 1.9480x over previous
import functools
import math

import jax
import jax.numpy as jnp
from jax.experimental import pallas as pl

_T = 8192
_D = 2048
_E = 64
_K = 8
_BLOCK_T = 2048
_EPS = 1e-5
_INV_SQRT2 = 1.0 / math.sqrt(2.0)


def _layernorm(h, gamma, beta):
    mu = jnp.mean(h, axis=-1, keepdims=True)
    var = jnp.mean((h - mu) ** 2, axis=-1, keepdims=True)
    return (h - mu) * jax.lax.rsqrt(var + _EPS) * gamma + beta


def _gelu_exact(h):
    return h * 0.5 * (1.0 + jax.lax.erf(h * _INV_SQRT2))


def _gate_kernel(x_ref, w1t_ref, b1_ref, g1_ref, be1_ref,
                 w2t_ref, b2_ref, g2_ref, be2_ref, w3t_ref,
                 w_ref, idx_ref, logits_ref):
    h = jnp.dot(x_ref[...], w1t_ref[...], preferred_element_type=jnp.float32)
    h = _gelu_exact(_layernorm(h + b1_ref[...], g1_ref[...], be1_ref[...]))
    h = jnp.dot(h, w2t_ref[...], preferred_element_type=jnp.float32)
    h = _gelu_exact(_layernorm(h + b2_ref[...], g2_ref[...], be2_ref[...]))
    logits = jnp.dot(h, w3t_ref[...], preferred_element_type=jnp.float32)
    logits_ref[...] = logits

    cols_i = jax.lax.broadcasted_iota(jnp.int32, logits.shape, 1)
    cols_desc = jnp.float32(_E - 1) - cols_i.astype(jnp.float32)
    work = logits
    vals = []
    idxs = []
    for _ in range(_K):
        m = jnp.max(work, axis=-1, keepdims=True)
        hit = work == m
        r = jnp.max(jnp.where(hit, cols_desc, -1.0), axis=-1, keepdims=True)
        vals.append(m)
        idxs.append(jnp.float32(_E - 1) - r)
        work = jnp.where(hit, -jnp.inf, work)
    v = jnp.concatenate(vals, axis=-1)
    idx_ref[...] = jnp.concatenate(idxs, axis=-1).astype(jnp.int32)
    e = jnp.exp(v - v[:, 0:1])
    w_ref[...] = e / jnp.sum(e, axis=-1, keepdims=True)


def kernel(x, w1, b1, g1, be1, w2, b2, g2, be2, w3):
    grid = (_T // _BLOCK_T,)
    tok = lambda i: (i, 0)
    rep = lambda i: (0, 0)
    out_shapes = (
        jax.ShapeDtypeStruct((_T, _K), jnp.float32),
        jax.ShapeDtypeStruct((_T, _K), jnp.int32),
        jax.ShapeDtypeStruct((_T, _E), jnp.float32),
    )
    f = pl.pallas_call(
        _gate_kernel,
        grid=grid,
        in_specs=[
            pl.BlockSpec((_BLOCK_T, _D), tok),
            pl.BlockSpec((_D, 128), rep),
            pl.BlockSpec((1, 128), rep),
            pl.BlockSpec((1, 128), rep),
            pl.BlockSpec((1, 128), rep),
            pl.BlockSpec((128, 32), rep),
            pl.BlockSpec((1, 32), rep),
            pl.BlockSpec((1, 32), rep),
            pl.BlockSpec((1, 32), rep),
            pl.BlockSpec((32, _E), rep),
        ],
        out_specs=(
            pl.BlockSpec((_BLOCK_T, _K), tok),
            pl.BlockSpec((_BLOCK_T, _K), tok),
            pl.BlockSpec((_BLOCK_T, _E), tok),
        ),
        out_shape=out_shapes,
    )
    return f(x, w1.T, b1[None, :], g1[None, :], be1[None, :],
             w2.T, b2[None, :], g2[None, :], be2[None, :], w3.T)


# speedup vs baseline: 1.9732x; 1.0129x over previous
import functools
import math

import jax
import jax.numpy as jnp
from jax.experimental import pallas as pl

_T = 8192
_D = 2048
_E = 64
_K = 8
_BLOCK_T = 1024
_EPS = 1e-5
_INV_SQRT2 = 1.0 / math.sqrt(2.0)


def _layernorm(h, gamma, beta):
    mu = jnp.mean(h, axis=-1, keepdims=True)
    var = jnp.mean((h - mu) ** 2, axis=-1, keepdims=True)
    return (h - mu) * jax.lax.rsqrt(var + _EPS) * gamma + beta


def _gelu_exact(h):
    return h * 0.5 * (1.0 + jax.lax.erf(h * _INV_SQRT2))


def _gate_kernel(x_ref, w1t_ref, b1_ref, g1_ref, be1_ref,
                 w2t_ref, b2_ref, g2_ref, be2_ref, w3t_ref,
                 w_ref, idx_ref, logits_ref):
    h = jnp.dot(x_ref[...], w1t_ref[...], preferred_element_type=jnp.float32)
    h = _gelu_exact(_layernorm(h + b1_ref[...], g1_ref[...], be1_ref[...]))
    h = jnp.dot(h, w2t_ref[...], preferred_element_type=jnp.float32)
    h = _gelu_exact(_layernorm(h + b2_ref[...], g2_ref[...], be2_ref[...]))
    logits = jnp.dot(h, w3t_ref[...], preferred_element_type=jnp.float32)
    logits_ref[...] = logits

    cols_i = jax.lax.broadcasted_iota(jnp.int32, logits.shape, 1)
    cols_desc = jnp.float32(_E - 1) - cols_i.astype(jnp.float32)
    work = logits
    vals = []
    idxs = []
    for _ in range(_K):
        m = jnp.max(work, axis=-1, keepdims=True)
        hit = work == m
        r = jnp.max(jnp.where(hit, cols_desc, -1.0), axis=-1, keepdims=True)
        vals.append(m)
        idxs.append(jnp.float32(_E - 1) - r)
        work = jnp.where(hit, -jnp.inf, work)
    v = jnp.concatenate(vals, axis=-1)
    idx_ref[...] = jnp.concatenate(idxs, axis=-1).astype(jnp.int32)
    e = jnp.exp(v - v[:, 0:1])
    w_ref[...] = e / jnp.sum(e, axis=-1, keepdims=True)


def kernel(x, w1, b1, g1, be1, w2, b2, g2, be2, w3):
    grid = (_T // _BLOCK_T,)
    tok = lambda i: (i, 0)
    rep = lambda i: (0, 0)
    out_shapes = (
        jax.ShapeDtypeStruct((_T, _K), jnp.float32),
        jax.ShapeDtypeStruct((_T, _K), jnp.int32),
        jax.ShapeDtypeStruct((_T, _E), jnp.float32),
    )
    f = pl.pallas_call(
        _gate_kernel,
        grid=grid,
        in_specs=[
            pl.BlockSpec((_BLOCK_T, _D), tok),
            pl.BlockSpec((_D, 128), rep),
            pl.BlockSpec((1, 128), rep),
            pl.BlockSpec((1, 128), rep),
            pl.BlockSpec((1, 128), rep),
            pl.BlockSpec((128, 32), rep),
            pl.BlockSpec((1, 32), rep),
            pl.BlockSpec((1, 32), rep),
            pl.BlockSpec((1, 32), rep),
            pl.BlockSpec((32, _E), rep),
        ],
        out_specs=(
            pl.BlockSpec((_BLOCK_T, _K), tok),
            pl.BlockSpec((_BLOCK_T, _K), tok),
            pl.BlockSpec((_BLOCK_T, _E), tok),
        ),
        out_shape=out_shapes,
    )
    return f(x, w1.T, b1[None, :], g1[None, :], be1[None, :],
             w2.T, b2[None, :], g2[None, :], be2[None, :], w3.T)
